# baseline (speedup 1.0000x reference)
import functools

import jax
import jax.numpy as jnp
from jax import lax
from jax.experimental import pallas as pl
from jax.experimental.pallas import tpu as pltpu
from jax.experimental.pallas import tpu_sc as plsc

D = 128
NUM_CORES = 2
NUM_SUBCORES = 16
NUM_WORKERS = NUM_CORES * NUM_SUBCORES
CHUNK = 64
NBUF = 4
EPS = 1e-5
LN_BLOCK = 512


def _gather_body(n_chunks, x2_hbm, table_hbm, out_hbm,
                 idx_all, rows_0, rows_1, rows_2, rows_3,
                 gsem_0, gsem_1, gsem_2, gsem_3,
                 ssem_0, ssem_1, ssem_2, ssem_3):
    wid = lax.axis_index("s") * NUM_CORES + lax.axis_index("c")
    rbase = wid * n_chunks * CHUNK

    rows = (rows_0, rows_1, rows_2, rows_3)
    gsem = (gsem_0, gsem_1, gsem_2, gsem_3)
    ssem = (ssem_0, ssem_1, ssem_2, ssem_3)

    pltpu.sync_copy(x2_hbm.at[wid], idx_all)

    def gather_desc(c, p):
        return pltpu.make_async_copy(table_hbm.at[idx_all.at[c]], rows[p],
                                     gsem[p])

    def scatter_desc(c, p):
        return pltpu.make_async_copy(
            rows[p], out_hbm.at[pl.ds(rbase + c * CHUNK, CHUNK)], ssem[p])

    gather_desc(0, 0).start()
    gather_desc(1, 1).start()
    gather_desc(2, 2).start()

    n_groups = n_chunks // NBUF

    def group_body(i, carry):
        for j in range(NBUF):
            c = NBUF * i + j
            pj = (j + 3) % NBUF
            gather_desc(c, j).wait()
            scatter_desc(c, j).start()

            @pl.when(c + 3 < n_chunks)
            def _():
                @pl.when(i + j > 0)
                def _():
                    scatter_desc(c - 1, pj).wait()
                gather_desc(c + 3, pj).start()

        return carry

    lax.fori_loop(0, n_groups, group_body, 0)

    for k in range(NBUF):
        c = n_chunks - NBUF + k
        scatter_desc(c, c % NBUF).wait()


def _sc_gather(x2, table, total):
    n_chunks = (total // NUM_WORKERS) // CHUNK
    assert n_chunks % NBUF == 0
    mesh = plsc.VectorSubcoreMesh(
        core_axis_name="c", subcore_axis_name="s",
        num_cores=NUM_CORES, num_subcores=NUM_SUBCORES)
    fn = pl.kernel(
        functools.partial(_gather_body, n_chunks),
        out_type=jax.ShapeDtypeStruct((total, D), jnp.float32),
        mesh=mesh,
        scratch_types=(
            [pltpu.VMEM((n_chunks, CHUNK), jnp.int32)]
            + [pltpu.VMEM((CHUNK, D), jnp.float32)] * NBUF
            + [pltpu.SemaphoreType.DMA] * (2 * NBUF)
        ),
        compiler_params=pltpu.CompilerParams(needs_layout_passes=False,
                                             use_tc_tiling_on_sc=True),
    )
    return fn(x2, table)


def _ln_block_kernel(emb_ref, lnw_ref, lnb_ref, out_ref):
    emb = emb_ref[...]
    mean = jnp.mean(emb, axis=-1, keepdims=True)
    cent = emb - mean
    var = jnp.mean(cent * cent, axis=-1, keepdims=True)
    rstd = lax.rsqrt(var + EPS)
    out_ref[...] = cent * rstd * lnw_ref[...] + lnb_ref[...]


def _tc_layernorm(emb, ln_w, ln_b, total):
    grid = total // LN_BLOCK
    return pl.pallas_call(
        _ln_block_kernel,
        grid=(grid,),
        in_specs=[
            pl.BlockSpec((LN_BLOCK, D), lambda i: (i, 0)),
            pl.BlockSpec((1, D), lambda i: (0, 0)),
            pl.BlockSpec((1, D), lambda i: (0, 0)),
        ],
        out_specs=pl.BlockSpec((LN_BLOCK, D), lambda i: (i, 0)),
        out_shape=jax.ShapeDtypeStruct((total, D), jnp.float32),
    )(emb, ln_w.reshape(1, D), ln_b.reshape(1, D))


def kernel(x, table, ln_w, ln_b):
    b, s = x.shape
    total = b * s
    assert total % (NUM_WORKERS * CHUNK) == 0
    n_chunks = (total // NUM_WORKERS) // CHUNK

    x2 = x.reshape(NUM_WORKERS, n_chunks, CHUNK)
    raw = _sc_gather(x2, table, total)
    out = _tc_layernorm(raw, ln_w, ln_b, total)
    return out.reshape(b, s, D)


# speedup vs baseline: 1.5423x; 1.5423x over previous
import functools

import jax
import jax.numpy as jnp
from jax import lax
from jax.experimental import pallas as pl
from jax.experimental.pallas import tpu as pltpu
from jax.experimental.pallas import tpu_sc as plsc

D = 128
NUM_CORES = 2
NUM_SUBCORES = 16
NUM_WORKERS = NUM_CORES * NUM_SUBCORES
CHUNK = 64
NBUF = 4
EPS = 1e-5
LN_BLOCK = 512


def _gather_body(n_chunks, x2_hbm, table_hbm, out_hbm,
                 idx_all, rows_0, rows_1, rows_2, rows_3,
                 gsem_0, gsem_1, gsem_2, gsem_3,
                 ssem_0, ssem_1, ssem_2, ssem_3):
    wid = lax.axis_index("s") * NUM_CORES + lax.axis_index("c")
    rbase = wid * n_chunks * CHUNK

    rows = (rows_0, rows_1, rows_2, rows_3)
    gsem = (gsem_0, gsem_1, gsem_2, gsem_3)
    ssem = (ssem_0, ssem_1, ssem_2, ssem_3)

    pltpu.sync_copy(x2_hbm.at[wid], idx_all)

    def gather_desc(c, p):
        return pltpu.make_async_copy(table_hbm.at[idx_all.at[c]], rows[p],
                                     gsem[p])

    def scatter_desc(c, p):
        return pltpu.make_async_copy(
            rows[p], out_hbm.at[pl.ds(rbase + c * CHUNK, CHUNK)], ssem[p])

    gather_desc(0, 0).start()
    gather_desc(1, 1).start()
    gather_desc(2, 2).start()

    n_groups = n_chunks // NBUF

    def group_body(i, carry):
        for j in range(NBUF):
            c = NBUF * i + j
            pj = (j + 3) % NBUF
            gather_desc(c, j).wait()
            scatter_desc(c, j).start()

            @pl.when(c + 3 < n_chunks)
            def _():
                @pl.when(i + j > 0)
                def _():
                    scatter_desc(c - 1, pj).wait()
                gather_desc(c + 3, pj).start()

        return carry

    lax.fori_loop(0, n_groups, group_body, 0)

    for k in range(NBUF):
        c = n_chunks - NBUF + k
        scatter_desc(c, c % NBUF).wait()


def _sc_gather(x2, table, total):
    n_chunks = (total // NUM_WORKERS) // CHUNK
    assert n_chunks % NBUF == 0
    mesh = plsc.VectorSubcoreMesh(
        core_axis_name="c", subcore_axis_name="s",
        num_cores=NUM_CORES, num_subcores=NUM_SUBCORES)
    fn = pl.kernel(
        functools.partial(_gather_body, n_chunks),
        out_type=jax.ShapeDtypeStruct((total, D), jnp.float32),
        mesh=mesh,
        scratch_types=(
            [pltpu.VMEM((n_chunks, CHUNK), jnp.int32)]
            + [pltpu.VMEM((CHUNK, D), jnp.float32)] * NBUF
            + [pltpu.SemaphoreType.DMA] * (2 * NBUF)
        ),
        compiler_params=pltpu.CompilerParams(needs_layout_passes=False,
                                             use_tc_tiling_on_sc=True),
    )
    return fn(x2, table)


def _ln_block_kernel(emb_ref, lnw_ref, lnb_ref, out_ref):
    emb = emb_ref[...]
    mean = jnp.mean(emb, axis=-1, keepdims=True)
    cent = emb - mean
    var = jnp.mean(cent * cent, axis=-1, keepdims=True)
    rstd = lax.rsqrt(var + EPS)
    out_ref[...] = cent * rstd * lnw_ref[...] + lnb_ref[...]


def _tc_layernorm(emb, ln_w, ln_b, total):
    grid = total // LN_BLOCK
    return pl.pallas_call(
        _ln_block_kernel,
        grid=(grid,),
        in_specs=[
            pl.BlockSpec((LN_BLOCK, D), lambda i: (i, 0)),
            pl.BlockSpec((1, D), lambda i: (0, 0)),
            pl.BlockSpec((1, D), lambda i: (0, 0)),
        ],
        out_specs=pl.BlockSpec((LN_BLOCK, D), lambda i: (i, 0)),
        out_shape=jax.ShapeDtypeStruct((total, D), jnp.float32),
    )(emb, ln_w.reshape(1, D), ln_b.reshape(1, D))


def kernel(x, table, ln_w, ln_b):
    b, s = x.shape
    total = b * s
    assert total % (NUM_WORKERS * CHUNK) == 0
    n_chunks = (total // NUM_WORKERS) // CHUNK

    xt = jnp.transpose(x).reshape(NUM_WORKERS, n_chunks, CHUNK)
    raw = _sc_gather(xt, table, total)
    out = _tc_layernorm(raw, ln_w, ln_b, total)
    return jnp.transpose(out.reshape(s, b, D), (1, 0, 2))


# speedup vs baseline: 2.0169x; 1.3077x over previous
import functools

import jax
import jax.numpy as jnp
from jax import lax
from jax.experimental import pallas as pl
from jax.experimental.pallas import tpu as pltpu
from jax.experimental.pallas import tpu_sc as plsc

D = 128
NUM_CORES = 2
NUM_SUBCORES = 16
NUM_WORKERS = NUM_CORES * NUM_SUBCORES
CHUNK = 64
NBUF = 4
EPS = 1e-5
LN_BLOCK = 1024


def _gather_body(n_chunks, x2_hbm, table_hbm, out_hbm,
                 idx_all, rows_0, rows_1, rows_2, rows_3,
                 gsem_0, gsem_1, gsem_2, gsem_3,
                 ssem_0, ssem_1, ssem_2, ssem_3):
    wid = lax.axis_index("s") * NUM_CORES + lax.axis_index("c")
    rbase = wid * n_chunks * CHUNK

    rows = (rows_0, rows_1, rows_2, rows_3)
    gsem = (gsem_0, gsem_1, gsem_2, gsem_3)
    ssem = (ssem_0, ssem_1, ssem_2, ssem_3)

    pltpu.sync_copy(x2_hbm.at[wid], idx_all)

    def gather_desc(c, p):
        return pltpu.make_async_copy(table_hbm.at[idx_all.at[c]], rows[p],
                                     gsem[p])

    def scatter_desc(c, p):
        return pltpu.make_async_copy(
            rows[p], out_hbm.at[pl.ds(rbase + c * CHUNK, CHUNK)], ssem[p])

    gather_desc(0, 0).start()
    gather_desc(1, 1).start()
    gather_desc(2, 2).start()

    n_groups = n_chunks // NBUF

    def group_body(i, carry):
        for j in range(NBUF):
            c = NBUF * i + j
            pj = (j + 3) % NBUF
            gather_desc(c, j).wait()
            scatter_desc(c, j).start()

            @pl.when(c + 3 < n_chunks)
            def _():
                @pl.when(i + j > 0)
                def _():
                    scatter_desc(c - 1, pj).wait()
                gather_desc(c + 3, pj).start()

        return carry

    lax.fori_loop(0, n_groups, group_body, 0)

    for k in range(NBUF):
        c = n_chunks - NBUF + k
        scatter_desc(c, c % NBUF).wait()


def _sc_gather(x2, table, total):
    n_chunks = (total // NUM_WORKERS) // CHUNK
    assert n_chunks % NBUF == 0
    mesh = plsc.VectorSubcoreMesh(
        core_axis_name="c", subcore_axis_name="s",
        num_cores=NUM_CORES, num_subcores=NUM_SUBCORES)
    fn = pl.kernel(
        functools.partial(_gather_body, n_chunks),
        out_type=jax.ShapeDtypeStruct((total, D), jnp.float32),
        mesh=mesh,
        scratch_types=(
            [pltpu.VMEM((n_chunks, CHUNK), jnp.int32)]
            + [pltpu.VMEM((CHUNK, D), jnp.float32)] * NBUF
            + [pltpu.SemaphoreType.DMA] * (2 * NBUF)
        ),
        compiler_params=pltpu.CompilerParams(needs_layout_passes=False,
                                             use_tc_tiling_on_sc=True),
    )
    return fn(x2, table)


def _ln_block_kernel(emb_ref, lnw_ref, lnb_ref, out_ref):
    emb = emb_ref[...]
    ones = jnp.full((D, 8), 1.0 / D, jnp.float32)
    mean8 = lax.dot_general(emb, ones, (((1,), (0,)), ((), ())),
                            preferred_element_type=jnp.float32)
    msq8 = lax.dot_general(emb * emb, ones, (((1,), (0,)), ((), ())),
                           preferred_element_type=jnp.float32)
    mean = mean8[:, :1]
    var = msq8[:, :1] - mean * mean
    rstd = lax.rsqrt(var + EPS)
    out_ref[...] = (emb - mean) * rstd * lnw_ref[...] + lnb_ref[...]


def _tc_layernorm(emb, ln_w, ln_b, total):
    grid = total // LN_BLOCK
    return pl.pallas_call(
        _ln_block_kernel,
        grid=(grid,),
        in_specs=[
            pl.BlockSpec((LN_BLOCK, D), lambda i: (i, 0)),
            pl.BlockSpec((1, D), lambda i: (0, 0)),
            pl.BlockSpec((1, D), lambda i: (0, 0)),
        ],
        out_specs=pl.BlockSpec((LN_BLOCK, D), lambda i: (i, 0)),
        out_shape=jax.ShapeDtypeStruct((total, D), jnp.float32),
    )(emb, ln_w.reshape(1, D), ln_b.reshape(1, D))


def kernel(x, table, ln_w, ln_b):
    b, s = x.shape
    total = b * s
    assert total % (NUM_WORKERS * CHUNK) == 0
    n_chunks = (total // NUM_WORKERS) // CHUNK

    xt = jnp.transpose(x).reshape(NUM_WORKERS, n_chunks, CHUNK)
    raw = _sc_gather(xt, table, total)
    out = _tc_layernorm(raw, ln_w, ln_b, total)
    return jnp.transpose(out.reshape(s, b, D), (1, 0, 2))


# speedup vs baseline: 2.1693x; 1.0756x over previous
import functools

import jax
import jax.numpy as jnp
from jax import lax
from jax.experimental import pallas as pl
from jax.experimental.pallas import tpu as pltpu
from jax.experimental.pallas import tpu_sc as plsc

D = 128
NUM_CORES = 2
NUM_SUBCORES = 16
NUM_WORKERS = NUM_CORES * NUM_SUBCORES
CHUNK = 64
NBUF = 4
EPS = 1e-5
LN_BLOCK = 1024


def _gather_body(n_chunks, x2_hbm, table_hbm, out_hbm,
                 idx_all, rows_0, rows_1, rows_2, rows_3,
                 gsem_0, gsem_1, gsem_2, gsem_3,
                 ssem_0, ssem_1, ssem_2, ssem_3):
    wid = lax.axis_index("s") * NUM_CORES + lax.axis_index("c")
    rbase = wid * n_chunks * CHUNK

    rows = (rows_0, rows_1, rows_2, rows_3)
    gsem = (gsem_0, gsem_1, gsem_2, gsem_3)
    ssem = (ssem_0, ssem_1, ssem_2, ssem_3)

    pltpu.sync_copy(x2_hbm.at[wid], idx_all)

    def gather_desc(c, p):
        return pltpu.make_async_copy(table_hbm.at[idx_all.at[c]], rows[p],
                                     gsem[p])

    def scatter_desc(c, p):
        return pltpu.make_async_copy(
            rows[p], out_hbm.at[pl.ds(rbase + c * CHUNK, CHUNK)], ssem[p])

    gather_desc(0, 0).start()
    gather_desc(1, 1).start()
    gather_desc(2, 2).start()

    n_groups = n_chunks // NBUF

    def group_body(i, carry):
        for j in range(NBUF):
            c = NBUF * i + j
            pj = (j + 3) % NBUF
            gather_desc(c, j).wait()
            scatter_desc(c, j).start()

            @pl.when(c + 3 < n_chunks)
            def _():
                @pl.when(i + j > 0)
                def _():
                    scatter_desc(c - 1, pj).wait()
                gather_desc(c + 3, pj).start()

        return carry

    lax.fori_loop(0, n_groups, group_body, 0)

    for k in range(NBUF):
        c = n_chunks - NBUF + k
        scatter_desc(c, c % NBUF).wait()


def _sc_gather(x2, table, total):
    n_chunks = (total // NUM_WORKERS) // CHUNK
    assert n_chunks % NBUF == 0
    mesh = plsc.VectorSubcoreMesh(
        core_axis_name="c", subcore_axis_name="s",
        num_cores=NUM_CORES, num_subcores=NUM_SUBCORES)
    fn = pl.kernel(
        functools.partial(_gather_body, n_chunks),
        out_type=jax.ShapeDtypeStruct((total, D), jnp.float32),
        mesh=mesh,
        scratch_types=(
            [pltpu.VMEM((n_chunks, CHUNK), jnp.int32)]
            + [pltpu.VMEM((CHUNK, D), jnp.float32)] * NBUF
            + [pltpu.SemaphoreType.DMA] * (2 * NBUF)
        ),
        compiler_params=pltpu.CompilerParams(needs_layout_passes=False,
                                             use_tc_tiling_on_sc=True),
    )
    return fn(x2, table)


def _ln_block_kernel(emb_ref, lnw_ref, lnb_ref, out_ref):
    emb = emb_ref[...]
    ones = jnp.full((D, D), 1.0 / D, jnp.float32)
    mean = lax.dot_general(emb, ones, (((1,), (0,)), ((), ())),
                           preferred_element_type=jnp.float32)
    msq = lax.dot_general(emb * emb, ones, (((1,), (0,)), ((), ())),
                          preferred_element_type=jnp.float32)
    var = msq - mean * mean
    rstd = lax.rsqrt(var + EPS)
    out_ref[...] = (emb - mean) * rstd * lnw_ref[...] + lnb_ref[...]


def _tc_layernorm(emb, ln_w, ln_b, total):
    grid = total // LN_BLOCK
    return pl.pallas_call(
        _ln_block_kernel,
        grid=(grid,),
        in_specs=[
            pl.BlockSpec((LN_BLOCK, D), lambda i: (i, 0)),
            pl.BlockSpec((1, D), lambda i: (0, 0)),
            pl.BlockSpec((1, D), lambda i: (0, 0)),
        ],
        out_specs=pl.BlockSpec((LN_BLOCK, D), lambda i: (i, 0)),
        out_shape=jax.ShapeDtypeStruct((total, D), jnp.float32),
    )(emb, ln_w.reshape(1, D), ln_b.reshape(1, D))


def kernel(x, table, ln_w, ln_b):
    b, s = x.shape
    total = b * s
    assert total % (NUM_WORKERS * CHUNK) == 0
    n_chunks = (total // NUM_WORKERS) // CHUNK

    xt = jnp.transpose(x).reshape(NUM_WORKERS, n_chunks, CHUNK)
    raw = _sc_gather(xt, table, total)
    out = _tc_layernorm(raw, ln_w, ln_b, total)
    return jnp.transpose(out.reshape(s, b, D), (1, 0, 2))


# speedup vs baseline: 3.1387x; 1.4468x over previous
import functools

import jax
import jax.numpy as jnp
from jax import lax
from jax.experimental import pallas as pl
from jax.experimental.pallas import tpu as pltpu
from jax.experimental.pallas import tpu_sc as plsc

D = 128
NUM_CORES = 2
NUM_SUBCORES = 16
NUM_WORKERS = NUM_CORES * NUM_SUBCORES
CHUNK = 64
NBUF = 4
EPS = 1e-5
LN_BLOCK = 4096


def _gather_body(n_chunks, x2_hbm, table_hbm, out_hbm,
                 idx_all, rows_0, rows_1, rows_2, rows_3,
                 gsem_0, gsem_1, gsem_2, gsem_3,
                 ssem_0, ssem_1, ssem_2, ssem_3):
    wid = lax.axis_index("s") * NUM_CORES + lax.axis_index("c")
    rbase = wid * n_chunks * CHUNK

    rows = (rows_0, rows_1, rows_2, rows_3)
    gsem = (gsem_0, gsem_1, gsem_2, gsem_3)
    ssem = (ssem_0, ssem_1, ssem_2, ssem_3)

    pltpu.sync_copy(x2_hbm.at[wid], idx_all)

    def gather_desc(c, p):
        return pltpu.make_async_copy(table_hbm.at[idx_all.at[c]], rows[p],
                                     gsem[p])

    def scatter_desc(c, p):
        return pltpu.make_async_copy(
            rows[p], out_hbm.at[pl.ds(rbase + c * CHUNK, CHUNK)], ssem[p])

    gather_desc(0, 0).start()
    gather_desc(1, 1).start()
    gather_desc(2, 2).start()

    n_groups = n_chunks // NBUF

    def group_body(i, carry):
        for j in range(NBUF):
            c = NBUF * i + j
            pj = (j + 3) % NBUF
            gather_desc(c, j).wait()
            scatter_desc(c, j).start()

            @pl.when(c + 3 < n_chunks)
            def _():
                @pl.when(i + j > 0)
                def _():
                    scatter_desc(c - 1, pj).wait()
                gather_desc(c + 3, pj).start()

        return carry

    lax.fori_loop(0, n_groups, group_body, 0)

    for k in range(NBUF):
        c = n_chunks - NBUF + k
        scatter_desc(c, c % NBUF).wait()


def _sc_gather(x2, table, total):
    n_chunks = (total // NUM_WORKERS) // CHUNK
    assert n_chunks % NBUF == 0
    mesh = plsc.VectorSubcoreMesh(
        core_axis_name="c", subcore_axis_name="s",
        num_cores=NUM_CORES, num_subcores=NUM_SUBCORES)
    fn = pl.kernel(
        functools.partial(_gather_body, n_chunks),
        out_type=jax.ShapeDtypeStruct((total, D), jnp.float32),
        mesh=mesh,
        scratch_types=(
            [pltpu.VMEM((n_chunks, CHUNK), jnp.int32)]
            + [pltpu.VMEM((CHUNK, D), jnp.float32)] * NBUF
            + [pltpu.SemaphoreType.DMA] * (2 * NBUF)
        ),
        compiler_params=pltpu.CompilerParams(needs_layout_passes=False,
                                             use_tc_tiling_on_sc=True),
    )
    return fn(x2, table)


def _ln_block_kernel(emb_ref, lnw_ref, lnb_ref, out_ref):
    emb = emb_ref[...]
    ones = jnp.full((D, D), 1.0 / D, jnp.float32)
    mean = lax.dot_general(emb, ones, (((1,), (0,)), ((), ())),
                           preferred_element_type=jnp.float32)
    msq = lax.dot_general(emb * emb, ones, (((1,), (0,)), ((), ())),
                          preferred_element_type=jnp.float32)
    var = msq - mean * mean
    rstd = lax.rsqrt(var + EPS)
    out_ref[...] = (emb - mean) * rstd * lnw_ref[...] + lnb_ref[...]


def _tc_layernorm(emb, ln_w, ln_b, total):
    grid = total // LN_BLOCK
    return pl.pallas_call(
        _ln_block_kernel,
        grid=(grid,),
        in_specs=[
            pl.BlockSpec((LN_BLOCK, D), lambda i: (i, 0)),
            pl.BlockSpec((1, D), lambda i: (0, 0)),
            pl.BlockSpec((1, D), lambda i: (0, 0)),
        ],
        out_specs=pl.BlockSpec((LN_BLOCK, D), lambda i: (i, 0)),
        out_shape=jax.ShapeDtypeStruct((total, D), jnp.float32),
    )(emb, ln_w.reshape(1, D), ln_b.reshape(1, D))


def kernel(x, table, ln_w, ln_b):
    b, s = x.shape
    total = b * s
    assert total % (NUM_WORKERS * CHUNK) == 0
    n_chunks = (total // NUM_WORKERS) // CHUNK

    xt = jnp.transpose(x).reshape(NUM_WORKERS, n_chunks, CHUNK)
    raw = _sc_gather(xt, table, total)
    out = _tc_layernorm(raw, ln_w, ln_b, total)
    return jnp.transpose(out.reshape(s, b, D), (1, 0, 2))


# speedup vs baseline: 3.4433x; 1.0970x over previous
import functools

import jax
import jax.numpy as jnp
from jax import lax
from jax.experimental import pallas as pl
from jax.experimental.pallas import tpu as pltpu
from jax.experimental.pallas import tpu_sc as plsc

D = 128
NUM_CORES = 2
NUM_SUBCORES = 16
NUM_WORKERS = NUM_CORES * NUM_SUBCORES
CHUNK = 64
NBUF = 4
EPS = 1e-5
LN_BLOCK = 8192


def _gather_body(n_chunks, x2_hbm, table_hbm, out_hbm,
                 idx_all, rows_0, rows_1, rows_2, rows_3,
                 gsem_0, gsem_1, gsem_2, gsem_3,
                 ssem_0, ssem_1, ssem_2, ssem_3):
    wid = lax.axis_index("s") * NUM_CORES + lax.axis_index("c")
    rbase = wid * n_chunks * CHUNK

    rows = (rows_0, rows_1, rows_2, rows_3)
    gsem = (gsem_0, gsem_1, gsem_2, gsem_3)
    ssem = (ssem_0, ssem_1, ssem_2, ssem_3)

    pltpu.sync_copy(x2_hbm.at[wid], idx_all)

    def gather_desc(c, p):
        return pltpu.make_async_copy(table_hbm.at[idx_all.at[c]], rows[p],
                                     gsem[p])

    def scatter_desc(c, p):
        return pltpu.make_async_copy(
            rows[p], out_hbm.at[pl.ds(rbase + c * CHUNK, CHUNK)], ssem[p])

    gather_desc(0, 0).start()
    gather_desc(1, 1).start()
    gather_desc(2, 2).start()

    n_groups = n_chunks // NBUF

    def group_body(i, carry):
        for j in range(NBUF):
            c = NBUF * i + j
            pj = (j + 3) % NBUF
            gather_desc(c, j).wait()
            scatter_desc(c, j).start()

            @pl.when(c + 3 < n_chunks)
            def _():
                @pl.when(i + j > 0)
                def _():
                    scatter_desc(c - 1, pj).wait()
                gather_desc(c + 3, pj).start()

        return carry

    lax.fori_loop(0, n_groups, group_body, 0)

    for k in range(NBUF):
        c = n_chunks - NBUF + k
        scatter_desc(c, c % NBUF).wait()


def _sc_gather(x2, table, total):
    n_chunks = (total // NUM_WORKERS) // CHUNK
    assert n_chunks % NBUF == 0
    mesh = plsc.VectorSubcoreMesh(
        core_axis_name="c", subcore_axis_name="s",
        num_cores=NUM_CORES, num_subcores=NUM_SUBCORES)
    fn = pl.kernel(
        functools.partial(_gather_body, n_chunks),
        out_type=jax.ShapeDtypeStruct((total, D), jnp.float32),
        mesh=mesh,
        scratch_types=(
            [pltpu.VMEM((n_chunks, CHUNK), jnp.int32)]
            + [pltpu.VMEM((CHUNK, D), jnp.float32)] * NBUF
            + [pltpu.SemaphoreType.DMA] * (2 * NBUF)
        ),
        compiler_params=pltpu.CompilerParams(needs_layout_passes=False,
                                             use_tc_tiling_on_sc=True),
    )
    return fn(x2, table)


def _ln_block_kernel(emb_ref, lnw_ref, lnb_ref, out_ref):
    emb = emb_ref[...]
    ones = jnp.full((D, D), 1.0 / D, jnp.float32)
    mean = lax.dot_general(emb, ones, (((1,), (0,)), ((), ())),
                           preferred_element_type=jnp.float32)
    msq = lax.dot_general(emb * emb, ones, (((1,), (0,)), ((), ())),
                          preferred_element_type=jnp.float32)
    var = msq - mean * mean
    rstd = lax.rsqrt(var + EPS)
    out_ref[...] = (emb - mean) * rstd * lnw_ref[...] + lnb_ref[...]


def _tc_layernorm(emb, ln_w, ln_b, total):
    grid = total // LN_BLOCK
    return pl.pallas_call(
        _ln_block_kernel,
        grid=(grid,),
        in_specs=[
            pl.BlockSpec((LN_BLOCK, D), lambda i: (i, 0)),
            pl.BlockSpec((1, D), lambda i: (0, 0)),
            pl.BlockSpec((1, D), lambda i: (0, 0)),
        ],
        out_specs=pl.BlockSpec((LN_BLOCK, D), lambda i: (i, 0)),
        out_shape=jax.ShapeDtypeStruct((total, D), jnp.float32),
    )(emb, ln_w.reshape(1, D), ln_b.reshape(1, D))


def kernel(x, table, ln_w, ln_b):
    b, s = x.shape
    total = b * s
    assert total % (NUM_WORKERS * CHUNK) == 0
    n_chunks = (total // NUM_WORKERS) // CHUNK

    xt = jnp.transpose(x).reshape(NUM_WORKERS, n_chunks, CHUNK)
    raw = _sc_gather(xt, table, total)
    out = _tc_layernorm(raw, ln_w, ln_b, total)
    return jnp.transpose(out.reshape(s, b, D), (1, 0, 2))


# speedup vs baseline: 4.3754x; 1.2707x over previous
import functools

import jax
import jax.numpy as jnp
from jax import lax
from jax.experimental import pallas as pl
from jax.experimental.pallas import tpu as pltpu
from jax.experimental.pallas import tpu_sc as plsc

D = 128
NUM_CORES = 2
NUM_SUBCORES = 16
NUM_WORKERS = NUM_CORES * NUM_SUBCORES
CHUNK = 64
NBUF = 4
EPS = 1e-5
LN_BLOCK = 8192


def _gather_body(n_chunks, x2_hbm, table_hbm, out_hbm,
                 idx_all, rows_0, rows_1, rows_2, rows_3,
                 gsem_0, gsem_1, gsem_2, gsem_3,
                 ssem_0, ssem_1, ssem_2, ssem_3):
    wid = lax.axis_index("s") * NUM_CORES + lax.axis_index("c")
    rbase = wid * n_chunks * CHUNK

    rows = (rows_0, rows_1, rows_2, rows_3)
    gsem = (gsem_0, gsem_1, gsem_2, gsem_3)
    ssem = (ssem_0, ssem_1, ssem_2, ssem_3)

    pltpu.sync_copy(x2_hbm.at[wid], idx_all)

    def gather_desc(c, p):
        return pltpu.make_async_copy(table_hbm.at[idx_all.at[c]], rows[p],
                                     gsem[p])

    def scatter_desc(c, p):
        return pltpu.make_async_copy(
            rows[p], out_hbm.at[pl.ds(rbase + c * CHUNK, CHUNK)], ssem[p])

    gather_desc(0, 0).start()
    gather_desc(1, 1).start()
    gather_desc(2, 2).start()

    n_groups = n_chunks // NBUF

    def group_body(i, carry):
        for j in range(NBUF):
            c = NBUF * i + j
            pj = (j + 3) % NBUF
            gather_desc(c, j).wait()
            scatter_desc(c, j).start()

            @pl.when(c + 3 < n_chunks)
            def _():
                @pl.when(i + j > 0)
                def _():
                    scatter_desc(c - 1, pj).wait()
                gather_desc(c + 3, pj).start()

        return carry

    lax.fori_loop(0, n_groups, group_body, 0)

    for k in range(NBUF):
        c = n_chunks - NBUF + k
        scatter_desc(c, c % NBUF).wait()


def _sc_gather(x2, table, total):
    n_chunks = (total // NUM_WORKERS) // CHUNK
    assert n_chunks % NBUF == 0
    mesh = plsc.VectorSubcoreMesh(
        core_axis_name="c", subcore_axis_name="s",
        num_cores=NUM_CORES, num_subcores=NUM_SUBCORES)
    fn = pl.kernel(
        functools.partial(_gather_body, n_chunks),
        out_type=jax.ShapeDtypeStruct((total, D), jnp.float32),
        mesh=mesh,
        scratch_types=(
            [pltpu.VMEM((n_chunks, CHUNK), jnp.int32)]
            + [pltpu.VMEM((CHUNK, D), jnp.float32)] * NBUF
            + [pltpu.SemaphoreType.DMA] * (2 * NBUF)
        ),
        compiler_params=pltpu.CompilerParams(needs_layout_passes=False,
                                             use_tc_tiling_on_sc=True),
    )
    return fn(x2, table)


def _ln_block_kernel(emb_ref, lnw_ref, lnb_ref, out_ref):
    emb = emb_ref[...]
    ones = jnp.full((D, D), 1.0 / D, jnp.float32)
    mean = lax.dot_general(emb, ones, (((1,), (0,)), ((), ())),
                           preferred_element_type=jnp.float32)
    msq = lax.dot_general(emb * emb, ones, (((1,), (0,)), ((), ())),
                          preferred_element_type=jnp.float32)
    var = msq - mean * mean
    rstd = lax.rsqrt(var + EPS)
    out_ref[...] = (emb - mean) * rstd * lnw_ref[...] + lnb_ref[...]


def _tc_layernorm(emb, ln_w, ln_b, nrows, block):
    grid = nrows // block
    return pl.pallas_call(
        _ln_block_kernel,
        grid=(grid,),
        in_specs=[
            pl.BlockSpec((block, D), lambda i: (i, 0)),
            pl.BlockSpec((1, D), lambda i: (0, 0)),
            pl.BlockSpec((1, D), lambda i: (0, 0)),
        ],
        out_specs=pl.BlockSpec((block, D), lambda i: (i, 0)),
        out_shape=jax.ShapeDtypeStruct((nrows, D), jnp.float32),
    )(emb, ln_w.reshape(1, D), ln_b.reshape(1, D))


def kernel(x, table, ln_w, ln_b):
    b, s = x.shape
    total = b * s
    assert total % (NUM_WORKERS * CHUNK) == 0
    n_chunks = (total // NUM_WORKERS) // CHUNK

    xt = jnp.transpose(x).reshape(NUM_WORKERS, n_chunks, CHUNK)
    v, _ = table.shape
    normed = _tc_layernorm(table, ln_w, ln_b, v, 10000)
    out = _sc_gather(xt, normed, total)
    return jnp.transpose(out.reshape(s, b, D), (1, 0, 2))
